# Initial kernel scaffold; baseline (speedup 1.0000x reference)
import jax
import jax.numpy as jnp
from jax.experimental import pallas as pl


def kernel(cls, label_cls, label_loc, pred_bboxes, label_target, dataset_id):
    raise NotImplementedError("write your pallas kernel here")


# speedup vs baseline: 1669.9534x; 1669.9534x over previous
import jax
import jax.numpy as jnp
from jax.experimental import pallas as pl
from jax.experimental.pallas import tpu as pltpu

_G1 = 3.0
_G2 = 3.0


def _elementwise(mask_i, cls1, llx, lly, bx1, by1, bx2, by2, lt):
    tx1, ty1, tx2, ty2 = lt[0], lt[1], lt[2], lt[3]
    mask = mask_i > 0
    p = jnp.exp(cls1)
    xx1 = jnp.maximum(tx1, bx1)
    yy1 = jnp.maximum(ty1, by1)
    xx2 = jnp.minimum(tx2, bx2)
    yy2 = jnp.minimum(ty2, by2)
    ww = jnp.maximum(xx2 - xx1, 0.0)
    hh = jnp.maximum(yy2 - yy1, 0.0)
    area = (bx2 - bx1) * (by2 - by1)
    ta = (tx2 - tx1) * (ty2 - ty1)
    inter = ww * hh
    iou = inter / (area + ta - inter)
    cx = llx + tx1
    cy = lly + ty1
    tcx = (tx1 + tx2) / 2.0
    tcy = (ty1 + ty2) / 2.0
    dist = jnp.sqrt((cx - tcx) ** 2 + (cy - tcy) ** 2)
    return mask, p, iou, dist


def _rank_loss_kernel(lc_r, cls_r, ll_r, pb_r, lc_c, cls_c, ll_c, pb_c,
                      lt_ref, did_ref, l1_ref, l2_ref, acc_ref):
    b = pl.program_id(0)
    nb = pl.num_programs(0)

    @pl.when(b == 0)
    def _init():
        acc_ref[0] = 0.0
        acc_ref[1] = 0.0
        acc_ref[2] = 0.0

    lt = (lt_ref[0, 0, 0], lt_ref[0, 0, 1], lt_ref[0, 0, 2], lt_ref[0, 0, 3])

    mask_r, p_r, iou_r, dist_r = _elementwise(
        lc_r[0], cls_r[0, 1:2, :], ll_r[0, 0:1, :], ll_r[0, 1:2, :],
        pb_r[0, 0:1, :], pb_r[0, 1:2, :], pb_r[0, 2:3, :], pb_r[0, 3:4, :], lt)
    mask_c, p_c, iou_c, dist_c = _elementwise(
        lc_c[0], cls_c[0, :, 1:2], ll_c[0, :, 0:1], ll_c[0, :, 1:2],
        pb_c[0, :, 0:1], pb_c[0, :, 1:2], pb_c[0, :, 2:3], pb_c[0, :, 3:4], lt)

    pm = mask_c & mask_r & (dist_r - dist_c >= 1.0)
    cnt = jnp.sum(pm.astype(jnp.float32))
    v1 = jnp.where(pm, jnp.exp(-_G1 * (p_c - p_r)), 0.0)
    v2 = jnp.where(pm, jnp.exp(-_G2 * (iou_c - iou_r)), 0.0)
    loss1 = jnp.sum(v1) / cnt
    loss2 = jnp.sum(v2) / cnt

    did = did_ref[0, 0, 0]
    valid = ((did != 1) & (cnt > 0.0)
             & jnp.logical_not(jnp.isnan(loss1))
             & jnp.logical_not(jnp.isnan(loss2)))
    acc_ref[0] += jnp.where(valid, loss1, 0.0)
    acc_ref[1] += jnp.where(valid, loss2, 0.0)
    acc_ref[2] += jnp.where(valid, 1.0, 0.0)

    @pl.when(b == nb - 1)
    def _fini():
        nv = acc_ref[2]
        l1_ref[0, 0] = jnp.where(nv > 0.0, acc_ref[0] / nv, 0.0)
        l2_ref[0, 0] = jnp.where(nv > 0.0, acc_ref[1] / nv, 0.0)


def kernel(cls, label_cls, label_loc, pred_bboxes, label_target, dataset_id):
    B = label_cls.shape[0]
    N = label_cls.shape[2] * label_cls.shape[3]
    cls_c = jnp.reshape(cls, (B, N, 2))
    cls_r = jnp.transpose(cls_c, (0, 2, 1))
    lc_r = jnp.reshape(label_cls, (B, 1, N))
    lc_c = jnp.transpose(lc_r, (0, 2, 1))
    ll_r = jnp.reshape(label_loc, (B, 4, N))
    ll_c = jnp.transpose(ll_r, (0, 2, 1))
    pb_r = pred_bboxes
    pb_c = jnp.transpose(pred_bboxes, (0, 2, 1))
    lt = jnp.reshape(label_target, (B, 1, 4))
    did = jnp.reshape(dataset_id, (B, 1, 1))

    def vspec(shape):
        nd = len(shape)
        return pl.BlockSpec((1,) + shape,
                            lambda b: (b,) + (0,) * nd)

    l1, l2 = pl.pallas_call(
        _rank_loss_kernel,
        grid=(B,),
        in_specs=[
            vspec((1, N)), vspec((2, N)), vspec((4, N)), vspec((4, N)),
            vspec((N, 1)), vspec((N, 2)), vspec((N, 4)), vspec((N, 4)),
            pl.BlockSpec((1, 1, 4), lambda b: (b, 0, 0), memory_space=pltpu.SMEM),
            pl.BlockSpec((1, 1, 1), lambda b: (b, 0, 0), memory_space=pltpu.SMEM),
        ],
        out_specs=[
            pl.BlockSpec((1, 1), lambda b: (0, 0), memory_space=pltpu.SMEM),
            pl.BlockSpec((1, 1), lambda b: (0, 0), memory_space=pltpu.SMEM),
        ],
        out_shape=[
            jax.ShapeDtypeStruct((1, 1), jnp.float32),
            jax.ShapeDtypeStruct((1, 1), jnp.float32),
        ],
        scratch_shapes=[pltpu.SMEM((4,), jnp.float32)],
    )(lc_r, cls_r, ll_r, pb_r, lc_c, cls_c, ll_c, pb_c, lt, did)
    return (l1[0, 0], l2[0, 0])


# speedup vs baseline: 2071.3702x; 1.2404x over previous
import functools

import jax
import jax.numpy as jnp
from jax import lax
from jax.experimental import pallas as pl
from jax.experimental.pallas import tpu as pltpu
from jax.experimental.pallas import tpu_sc as plsc

_G1 = 3.0
_G2 = 3.0
_PSHIFT = 15.0
_NPAD = 640
_NC = 2
_NS = 16
_CHUNKS = 4
_CVECS = _NPAD // (_CHUNKS * 16)


def _prep_kernel(lc_ref, cls_ref, ll_ref, pb_ref, lt_ref, pr_ref):
    mask = lc_ref[:, 0, :] > 0
    p = jnp.exp(cls_ref[:, 1, :])

    bx1 = pb_ref[:, 0, :]
    by1 = pb_ref[:, 1, :]
    bx2 = pb_ref[:, 2, :]
    by2 = pb_ref[:, 3, :]
    tx1 = lt_ref[:, 0:1]
    ty1 = lt_ref[:, 1:2]
    tx2 = lt_ref[:, 2:3]
    ty2 = lt_ref[:, 3:4]

    xx1 = jnp.maximum(tx1, bx1)
    yy1 = jnp.maximum(ty1, by1)
    xx2 = jnp.minimum(tx2, bx2)
    yy2 = jnp.minimum(ty2, by2)
    ww = jnp.maximum(xx2 - xx1, 0.0)
    hh = jnp.maximum(yy2 - yy1, 0.0)
    area = (bx2 - bx1) * (by2 - by1)
    ta = (tx2 - tx1) * (ty2 - ty1)
    inter = ww * hh
    iou = inter / (area + ta - inter)

    cx = ll_ref[:, 0, :] + tx1
    cy = ll_ref[:, 1, :] + ty1
    tcx = (tx1 + tx2) / 2.0
    tcy = (ty1 + ty2) / 2.0
    dist = jnp.sqrt((cx - tcx) ** 2 + (cy - tcy) ** 2)

    ps = p - _PSHIFT
    pr_ref[:, 0, :] = jnp.where(mask, dist, -1e30)
    pr_ref[:, 1, :] = jnp.exp(_G1 * ps)
    pr_ref[:, 2, :] = jnp.exp(_G2 * iou)
    pr_ref[:, 3, :] = jnp.where(mask, dist + 1.0, 1e30)
    pr_ref[:, 4, :] = jnp.where(mask, jnp.exp(-_G1 * ps), 0.0)
    pr_ref[:, 5, :] = jnp.where(mask, jnp.exp(-_G2 * iou), 0.0)


def _sc_pair_kernel(pr_hbm, out_hbm, buf, obuf):
    sample = lax.axis_index("s")
    half = lax.axis_index("c")

    pltpu.sync_copy(pr_hbm.at[sample], buf)

    abase = half * (_NPAD // 2)
    zero = jnp.zeros((16,), jnp.float32)
    t1, t2, t3 = zero, zero, zero
    for c in range(_CHUNKS):
        bd = [buf[0, pl.ds(c * _CVECS * 16 + j * 16, 16)] for j in range(_CVECS)]
        bp = [buf[1, pl.ds(c * _CVECS * 16 + j * 16, 16)] for j in range(_CVECS)]
        bi = [buf[2, pl.ds(c * _CVECS * 16 + j * 16, 16)] for j in range(_CVECS)]

        def body(k, carry, bd=bd, bp=bp, bi=bi):
            t1, t2, t3 = carry
            a0 = abase + k * 16
            tav = buf[3, pl.ds(a0, 16)]
            e1pv = buf[4, pl.ds(a0, 16)]
            e1iv = buf[5, pl.ds(a0, 16)]
            for l in range(16):
                ta = tav[l]
                v1 = v2 = v3 = zero
                for j in range(_CVECS):
                    m = bd[j] >= ta
                    v1 = v1 + jnp.where(m, bp[j], 0.0)
                    v2 = v2 + jnp.where(m, bi[j], 0.0)
                    v3 = v3 + jnp.where(m, 1.0, 0.0)
                t1 = t1 + e1pv[l] * v1
                t2 = t2 + e1iv[l] * v2
                t3 = t3 + v3
            return (t1, t2, t3)

        t1, t2, t3 = lax.fori_loop(0, _NPAD // 32, body, (t1, t2, t3))

    obuf[pl.ds(0, 16)] = t1
    obuf[pl.ds(16, 16)] = t2
    obuf[pl.ds(32, 16)] = t3
    pltpu.sync_copy(obuf, out_hbm.at[half * _NS + sample])


def _finalize_kernel(parts_ref, did_ref, l1_ref, l2_ref):
    s = parts_ref[0:_NS, :] + parts_ref[_NS:2 * _NS, :]
    s1 = jnp.sum(s[:, 0:16], axis=1, keepdims=True)
    s2 = jnp.sum(s[:, 16:32], axis=1, keepdims=True)
    cnt = jnp.sum(s[:, 32:48], axis=1, keepdims=True)
    did = did_ref[:, 0, :]
    valid = (did != 1) & (cnt > 0.0)
    vf = valid.astype(jnp.float32)
    l1 = jnp.where(valid, s1 / cnt, 0.0)
    l2 = jnp.where(valid, s2 / cnt, 0.0)
    nv = jnp.sum(vf)
    l1_ref[0, 0] = jnp.where(nv > 0.0, jnp.sum(l1) / nv, 0.0)
    l2_ref[0, 0] = jnp.where(nv > 0.0, jnp.sum(l2) / nv, 0.0)


def kernel(cls, label_cls, label_loc, pred_bboxes, label_target, dataset_id):
    B = label_cls.shape[0]
    N = label_cls.shape[2] * label_cls.shape[3]
    assert B == _NS and N <= _NPAD
    pad = _NPAD - N

    lc = jnp.pad(jnp.reshape(label_cls, (B, 1, N)), ((0, 0), (0, 0), (0, pad)))
    cls_t = jnp.pad(jnp.transpose(jnp.reshape(cls, (B, N, 2)), (0, 2, 1)),
                    ((0, 0), (0, 0), (0, pad)))
    ll = jnp.pad(jnp.reshape(label_loc, (B, 4, N)), ((0, 0), (0, 0), (0, pad)))
    pb = jnp.pad(pred_bboxes, ((0, 0), (0, 0), (0, pad)))
    lt = jnp.reshape(label_target, (B, 4))
    did = jnp.reshape(dataset_id, (B, 1, 1))

    pr = pl.pallas_call(
        _prep_kernel,
        in_specs=[
            pl.BlockSpec((B, 1, _NPAD), lambda: (0, 0, 0)),
            pl.BlockSpec((B, 2, _NPAD), lambda: (0, 0, 0)),
            pl.BlockSpec((B, 4, _NPAD), lambda: (0, 0, 0)),
            pl.BlockSpec((B, 4, _NPAD), lambda: (0, 0, 0)),
            pl.BlockSpec((B, 4), lambda: (0, 0)),
        ],
        out_specs=pl.BlockSpec((B, 6, _NPAD), lambda: (0, 0, 0)),
        out_shape=jax.ShapeDtypeStruct((B, 6, _NPAD), jnp.float32),
    )(lc, cls_t, ll, pb, lt)

    mesh = plsc.VectorSubcoreMesh(core_axis_name="c", subcore_axis_name="s",
                                  num_cores=_NC, num_subcores=_NS)
    parts = pl.kernel(
        _sc_pair_kernel,
        out_type=jax.ShapeDtypeStruct((_NC * _NS, 48), jnp.float32),
        mesh=mesh,
        scratch_types=[
            pltpu.VMEM((6, _NPAD), jnp.float32),
            pltpu.VMEM((48,), jnp.float32),
        ],
    )(pr)

    l1, l2 = pl.pallas_call(
        _finalize_kernel,
        in_specs=[
            pl.BlockSpec((_NC * _NS, 48), lambda: (0, 0)),
            pl.BlockSpec((B, 1, 1), lambda: (0, 0, 0)),
        ],
        out_specs=[
            pl.BlockSpec((1, 1), lambda: (0, 0), memory_space=pltpu.SMEM),
            pl.BlockSpec((1, 1), lambda: (0, 0), memory_space=pltpu.SMEM),
        ],
        out_shape=[
            jax.ShapeDtypeStruct((1, 1), jnp.float32),
            jax.ShapeDtypeStruct((1, 1), jnp.float32),
        ],
    )(parts, did)
    return (l1[0, 0], l2[0, 0])
